# Initial kernel scaffold; baseline (speedup 1.0000x reference)
import jax
import jax.numpy as jnp
from jax.experimental import pallas as pl


def kernel(xyz, new_xyz, features):
    raise NotImplementedError("write your pallas kernel here")


# speedup vs baseline: 19.9773x; 19.9773x over previous
import jax
import jax.numpy as jnp
from jax import lax
from jax.experimental import pallas as pl
from jax.experimental.pallas import tpu as pltpu
from jax.experimental.pallas import tpu_sc as plsc

_RADIUS = 0.2
_NSAMPLE = 64
_NC, _NS, _L = 2, 16, 16
_NW = _NC * _NS


def _round_bf16(v):
    u = plsc.bitcast(v, jnp.int32)
    r = lax.shift_right_logical(u, 16) & 1
    u2 = (u + 32767 + r) & jnp.int32(-65536)
    return plsc.bitcast(u2, jnp.float32)


def _make_sc_kernel(B, N, P, C):
    assert (B * P) % _NW == 0
    cpw = (B * P) // _NW
    wpb = _NW // B
    assert N % _L == 0 and _NSAMPLE % _L == 0
    n_chunks = N // _L
    r2 = jnp.float32(_RADIUS * _RADIUS)
    NS4 = _NSAMPLE // _L

    def body(xs, ys, zs, cxs, cys, czs, feat, out,
             xt, yt, zt, x2t, cbx, cby, cbz, cbuf, idx_all, staging):
        wid = lax.axis_index("s") * _NC + lax.axis_index("c")
        b = wid // wpb
        p0 = (wid % wpb) * cpw

        pltpu.sync_copy(xs.at[b], xt)
        pltpu.sync_copy(ys.at[b], yt)
        pltpu.sync_copy(zs.at[b], zt)
        pltpu.sync_copy(cxs.at[b, pl.ds(p0, cpw)], cbx)
        pltpu.sync_copy(cys.at[b, pl.ds(p0, cpw)], cby)
        pltpu.sync_copy(czs.at[b, pl.ds(p0, cpw)], cbz)

        iota = lax.iota(jnp.int32, _L)
        zero16 = jnp.zeros((_L,), jnp.int32)

        def x2_body(i, _):
            off = i * _L
            xv = xt[pl.ds(off, _L)]
            yv = yt[pl.ds(off, _L)]
            zv = zt[pl.ds(off, _L)]
            x2t[pl.ds(off, _L)] = (xv * xv + yv * yv) + zv * zv
            xt[pl.ds(off, _L)] = _round_bf16(xv)
            yt[pl.ds(off, _L)] = _round_bf16(yv)
            zt[pl.ds(off, _L)] = _round_bf16(zv)
            return 0
        lax.fori_loop(0, n_chunks, x2_body, 0)

        def center_body(p, _):
            pidx = jnp.full((_L,), p, jnp.int32)
            cxv = plsc.load_gather(cbx, [pidx])
            cyv = plsc.load_gather(cby, [pidx])
            czv = plsc.load_gather(cbz, [pidx])
            n2v = (cxv * cxv + cyv * cyv) + czv * czv
            cxb = _round_bf16(cxv)
            cyb = _round_bf16(cyv)
            czb = _round_bf16(czv)

            def cond(carry):
                j, cnt = carry
                return jnp.logical_and(cnt < _NSAMPLE, j < n_chunks)

            def wbody(carry):
                j, cnt = carry
                off = j * _L
                xv = xt[pl.ds(off, _L)]
                yv = yt[pl.ds(off, _L)]
                zv = zt[pl.ds(off, _L)]
                x2v = x2t[pl.ds(off, _L)]
                dot = (cxb * xv + cyb * yv) + czb * zv
                d2 = (n2v + x2v) - 2.0 * dot
                m = d2 < r2
                plsc.store_compressed(cbuf.at[pl.ds(cnt, _L)], iota + off,
                                      mask=m)
                return (j + 1, cnt + jnp.sum(m.astype(jnp.int32)))

            _, cnt = lax.while_loop(cond, wbody,
                                    (jnp.int32(0), jnp.int32(0)))
            total = jnp.minimum(cnt, _NSAMPLE)
            tsplat = jnp.full((_L,), total, jnp.int32)
            firstv = plsc.load_gather(cbuf, [zero16])
            firstv = jnp.where(tsplat > 0, firstv, 0)
            for k in range(NS4):
                sl = iota + k * _L
                v = cbuf[pl.ds(k * _L, _L)]
                v = jnp.where(sl < tsplat, v, firstv)
                idx_all[p, pl.ds(k * _L, _L)] = v
            return 0
        lax.fori_loop(0, cpw, center_body, 0)

        pltpu.sync_copy(xs.at[b], xt)
        pltpu.sync_copy(ys.at[b], yt)
        pltpu.sync_copy(zs.at[b], zt)

        for cdim in range(3):
            tab = (xt, yt, zt)[cdim]
            cb = (cbx, cby, cbz)[cdim]

            def gx_body(p, _, tab=tab, cb=cb):
                pidx = jnp.full((_L,), p, jnp.int32)
                cv = plsc.load_gather(cb, [pidx])
                for k in range(NS4):
                    idxv = idx_all[p, pl.ds(k * _L, _L)]
                    vals = plsc.load_gather(tab, [idxv])
                    staging[p, pl.ds(k * _L, _L)] = vals - cv
                return 0
            lax.fori_loop(0, cpw, gx_body, 0)
            pltpu.sync_copy(staging, out.at[b, cdim, pl.ds(p0, cpw), :])

        def chan_body(c, _):
            pltpu.sync_copy(feat.at[b, c], x2t)

            def g_body(p, _):
                for k in range(NS4):
                    idxv = idx_all[p, pl.ds(k * _L, _L)]
                    staging[p, pl.ds(k * _L, _L)] = plsc.load_gather(
                        x2t, [idxv])
                return 0
            lax.fori_loop(0, cpw, g_body, 0)
            pltpu.sync_copy(staging, out.at[b, 3 + c, pl.ds(p0, cpw), :])
            return 0
        lax.fori_loop(0, C, chan_body, 0)

    mesh = plsc.VectorSubcoreMesh(core_axis_name="c", subcore_axis_name="s",
                                  num_cores=_NC, num_subcores=_NS)
    return pl.kernel(
        body,
        out_type=jax.ShapeDtypeStruct((B, 3 + C, P, _NSAMPLE), jnp.float32),
        mesh=mesh,
        compiler_params=pltpu.CompilerParams(needs_layout_passes=False),
        scratch_types=[
            pltpu.VMEM((N,), jnp.float32),
            pltpu.VMEM((N,), jnp.float32),
            pltpu.VMEM((N,), jnp.float32),
            pltpu.VMEM((N,), jnp.float32),
            pltpu.VMEM((cpw,), jnp.float32),
            pltpu.VMEM((cpw,), jnp.float32),
            pltpu.VMEM((cpw,), jnp.float32),
            pltpu.VMEM((_NSAMPLE + _L,), jnp.int32),
            pltpu.VMEM((cpw, _NSAMPLE), jnp.int32),
            pltpu.VMEM((cpw, _NSAMPLE), jnp.float32),
        ],
    )


def kernel(xyz, new_xyz, features):
    B, N, _ = xyz.shape
    P = new_xyz.shape[1]
    C = features.shape[1]
    xs = xyz[:, :, 0]
    ys = xyz[:, :, 1]
    zs = xyz[:, :, 2]
    cxs = new_xyz[:, :, 0]
    cys = new_xyz[:, :, 1]
    czs = new_xyz[:, :, 2]
    run = _make_sc_kernel(B, N, P, C)
    return run(xs, ys, zs, cxs, cys, czs, features)


# speedup vs baseline: 47.7770x; 2.3916x over previous
import jax
import jax.numpy as jnp
from jax import lax
from jax.experimental import pallas as pl
from jax.experimental.pallas import tpu as pltpu
from jax.experimental.pallas import tpu_sc as plsc

_RADIUS = 0.2
_NSAMPLE = 64
_NC, _NS, _L = 2, 16, 16
_NW = _NC * _NS
_BLK = 4


def _round_bf16(v):
    u = plsc.bitcast(v, jnp.int32)
    r = lax.shift_right_logical(u, 16) & 1
    u2 = (u + 32767 + r) & jnp.int32(-65536)
    return plsc.bitcast(u2, jnp.float32)


def _make_sc_kernel(B, N, P, C):
    assert (B * P) % _NW == 0
    cpw = (B * P) // _NW
    wpb = _NW // B
    assert N % (_L * _BLK) == 0 and _NSAMPLE % _L == 0
    n_blocks = N // (_L * _BLK)
    r2 = jnp.float32(_RADIUS * _RADIUS)
    NS4 = _NSAMPLE // _L
    NCH = 3 + C

    def body(xyzt, cxs, cys, czs, feat, out,
             t1, t2, t3, t4, cball, cbuf, idx_all, sA, sB,
             semTA, semTB, semWA, semWB):
        wid = lax.axis_index("s") * _NC + lax.axis_index("c")
        b = wid // wpb
        p0 = (wid % wpb) * cpw

        pltpu.sync_copy(xyzt.at[3 * b + 0], t1)
        pltpu.sync_copy(xyzt.at[3 * b + 1], t2)
        pltpu.sync_copy(xyzt.at[3 * b + 2], t3)
        pltpu.sync_copy(cxs.at[b, pl.ds(p0, cpw)], cball.at[pl.ds(0, cpw)])
        pltpu.sync_copy(cys.at[b, pl.ds(p0, cpw)], cball.at[pl.ds(cpw, cpw)])
        pltpu.sync_copy(czs.at[b, pl.ds(p0, cpw)],
                        cball.at[pl.ds(2 * cpw, cpw)])

        iota = lax.iota(jnp.int32, _L)
        zero16 = jnp.zeros((_L,), jnp.int32)

        @plsc.parallel_loop(0, N, step=_L, unroll=2)
        def _(off):
            xv = t1[pl.ds(off, _L)]
            yv = t2[pl.ds(off, _L)]
            zv = t3[pl.ds(off, _L)]
            t4[pl.ds(off, _L)] = (xv * xv + yv * yv) + zv * zv
            t1[pl.ds(off, _L)] = _round_bf16(xv)
            t2[pl.ds(off, _L)] = _round_bf16(yv)
            t3[pl.ds(off, _L)] = _round_bf16(zv)

        def center_body(p, _):
            cxv = plsc.load_gather(cball, [jnp.full((_L,), p, jnp.int32)])
            cyv = plsc.load_gather(cball,
                                   [jnp.full((_L,), cpw + p, jnp.int32)])
            czv = plsc.load_gather(cball,
                                   [jnp.full((_L,), 2 * cpw + p, jnp.int32)])
            n2v = (cxv * cxv + cyv * cyv) + czv * czv
            cxb = _round_bf16(cxv)
            cyb = _round_bf16(cyv)
            czb = _round_bf16(czv)

            def cond(carry):
                j, cnt = carry
                return jnp.logical_and(cnt < _NSAMPLE, j < n_blocks)

            def wbody(carry):
                j, cnt = carry
                base = j * (_L * _BLK)
                masks = []
                sums = []
                for k in range(_BLK):
                    off = base + k * _L
                    xv = t1[pl.ds(off, _L)]
                    yv = t2[pl.ds(off, _L)]
                    zv = t3[pl.ds(off, _L)]
                    x2v = t4[pl.ds(off, _L)]
                    dot = (cxb * xv + cyb * yv) + czb * zv
                    d2 = (n2v + x2v) - 2.0 * dot
                    m = d2 < r2
                    masks.append(m)
                    sums.append(jnp.sum(m.astype(jnp.int32)))
                off_k = cnt
                for k in range(_BLK):
                    plsc.store_compressed(cbuf.at[pl.ds(off_k, _L)],
                                          iota + (base + k * _L),
                                          mask=masks[k])
                    off_k = off_k + sums[k]
                return (j + 1, off_k)

            _, cnt = lax.while_loop(cond, wbody,
                                    (jnp.int32(0), jnp.int32(0)))
            total = jnp.minimum(cnt, _NSAMPLE)
            tsplat = jnp.full((_L,), total, jnp.int32)
            firstv = plsc.load_gather(cbuf, [zero16])
            firstv = jnp.where(tsplat > 0, firstv, 0)
            for k in range(NS4):
                sl = iota + k * _L
                v = cbuf[pl.ds(k * _L, _L)]
                v = jnp.where(sl < tsplat, v, firstv)
                idx_all[p, pl.ds(k * _L, _L)] = v
            return 0
        lax.fori_loop(0, cpw, center_body, 0)

        def start_table(c, tab, sem):
            @pl.when(c < 3)
            def _():
                pltpu.async_copy(xyzt.at[3 * b + c], tab, sem)

            @pl.when(jnp.logical_and(c >= 3, c < NCH))
            def _():
                pltpu.async_copy(feat.at[b, c - 3], tab, sem)

        def wait_table(tab, sem):
            pltpu.make_async_copy(feat.at[b, 0], tab, sem).wait()

        def gather_chan(c, tab, stg):
            @pl.when(c < 3)
            def _():
                @plsc.parallel_loop(0, cpw, unroll=2)
                def _(p):
                    cidx = jnp.full((_L,), c * cpw + p, jnp.int32)
                    cv = plsc.load_gather(cball, [cidx])
                    for k in range(NS4):
                        idxv = idx_all[p, pl.ds(k * _L, _L)]
                        vals = plsc.load_gather(tab, [idxv])
                        stg[p, pl.ds(k * _L, _L)] = vals - cv

            @pl.when(c >= 3)
            def _():
                @plsc.parallel_loop(0, cpw, unroll=2)
                def _(p):
                    for k in range(NS4):
                        idxv = idx_all[p, pl.ds(k * _L, _L)]
                        stg[p, pl.ds(k * _L, _L)] = plsc.load_gather(
                            tab, [idxv])

        def half(i, c, tab, stg, semT, semW):
            wait_table(tab, semT)

            @pl.when(i > 0)
            def _():
                pltpu.make_async_copy(
                    stg, out.at[b, c - 2, pl.ds(p0, cpw), :], semW).wait()
            gather_chan(c, tab, stg)
            start_table(c + 2, tab, semT)
            pltpu.async_copy(stg, out.at[b, c, pl.ds(p0, cpw), :], semW)

        start_table(jnp.int32(0), t1, semTA)
        start_table(jnp.int32(1), t2, semTB)

        npairs = NCH // 2

        def chan_pair(i, _):
            half(i, 2 * i, t1, sA, semTA, semWA)
            half(i, 2 * i + 1, t2, sB, semTB, semWB)
            return 0
        lax.fori_loop(0, npairs, chan_pair, 0)

        if NCH % 2:
            cR = jnp.int32(NCH - 1)
            wait_table(t1, semTA)
            pltpu.make_async_copy(
                sA, out.at[b, cR - 2, pl.ds(p0, cpw), :], semWA).wait()
            gather_chan(cR, t1, sA)
            pltpu.async_copy(sA, out.at[b, cR, pl.ds(p0, cpw), :], semWA)
            pltpu.make_async_copy(
                sB, out.at[b, cR - 1, pl.ds(p0, cpw), :], semWB).wait()
            pltpu.make_async_copy(
                sA, out.at[b, cR, pl.ds(p0, cpw), :], semWA).wait()
        else:
            pltpu.make_async_copy(
                sA, out.at[b, NCH - 2, pl.ds(p0, cpw), :], semWA).wait()
            pltpu.make_async_copy(
                sB, out.at[b, NCH - 1, pl.ds(p0, cpw), :], semWB).wait()

    mesh = plsc.VectorSubcoreMesh(core_axis_name="c", subcore_axis_name="s",
                                  num_cores=_NC, num_subcores=_NS)
    return pl.kernel(
        body,
        out_type=jax.ShapeDtypeStruct((B, NCH, P, _NSAMPLE), jnp.float32),
        mesh=mesh,
        compiler_params=pltpu.CompilerParams(needs_layout_passes=False),
        scratch_types=[
            pltpu.VMEM((N,), jnp.float32),
            pltpu.VMEM((N,), jnp.float32),
            pltpu.VMEM((N,), jnp.float32),
            pltpu.VMEM((N,), jnp.float32),
            pltpu.VMEM((3 * cpw,), jnp.float32),
            pltpu.VMEM((_NSAMPLE + _L * _BLK,), jnp.int32),
            pltpu.VMEM((cpw, _NSAMPLE), jnp.int32),
            pltpu.VMEM((cpw, _NSAMPLE), jnp.float32),
            pltpu.VMEM((cpw, _NSAMPLE), jnp.float32),
            pltpu.SemaphoreType.DMA,
            pltpu.SemaphoreType.DMA,
            pltpu.SemaphoreType.DMA,
            pltpu.SemaphoreType.DMA,
        ],
    )


def kernel(xyz, new_xyz, features):
    B, N, _ = xyz.shape
    P = new_xyz.shape[1]
    C = features.shape[1]
    xyzt = jnp.transpose(xyz, (0, 2, 1)).reshape(B * 3, N)
    cxs = new_xyz[:, :, 0]
    cys = new_xyz[:, :, 1]
    czs = new_xyz[:, :, 2]
    run = _make_sc_kernel(B, N, P, C)
    return run(xyzt, cxs, cys, czs, features)


# speedup vs baseline: 71.8555x; 1.5040x over previous
import jax
import jax.numpy as jnp
from jax import lax
from jax.experimental import pallas as pl
from jax.experimental.pallas import tpu as pltpu
from jax.experimental.pallas import tpu_sc as plsc

_RADIUS = 0.2
_NSAMPLE = 64
_NC, _NS, _L = 2, 16, 16
_NW = _NC * _NS
_BLK = 4


def _round_bf16(v):
    u = plsc.bitcast(v, jnp.int32)
    r = lax.shift_right_logical(u, 16) & 1
    u2 = (u + 32767 + r) & jnp.int32(-65536)
    return plsc.bitcast(u2, jnp.float32)


def _make_sc_kernel(B, N, P, C):
    assert (B * P) % _NW == 0
    cpw = (B * P) // _NW
    wpb = _NW // B
    assert N % (_L * _BLK) == 0 and _NSAMPLE % _L == 0
    n_blocks = N // (_L * _BLK)
    r2 = jnp.float32(_RADIUS * _RADIUS)
    NS4 = _NSAMPLE // _L
    NCH = 3 + C

    def body(xyzt, cxs, cys, czs, feat, out,
             t1, t2, t3, t4, cball, cbuf, idx_all, sA, sB,
             semTA, semTB, semWA, semWB):
        wid = lax.axis_index("s") * _NC + lax.axis_index("c")
        b = wid // wpb
        p0 = (wid % wpb) * cpw

        pltpu.sync_copy(xyzt.at[3 * b + 0], t1)
        pltpu.sync_copy(xyzt.at[3 * b + 1], t2)
        pltpu.sync_copy(xyzt.at[3 * b + 2], t3)
        pltpu.sync_copy(cxs.at[b, pl.ds(p0, cpw)], cball.at[pl.ds(0, cpw)])
        pltpu.sync_copy(cys.at[b, pl.ds(p0, cpw)], cball.at[pl.ds(cpw, cpw)])
        pltpu.sync_copy(czs.at[b, pl.ds(p0, cpw)],
                        cball.at[pl.ds(2 * cpw, cpw)])

        iota = lax.iota(jnp.int32, _L)
        zero16 = jnp.zeros((_L,), jnp.int32)

        @plsc.parallel_loop(0, N, step=_L, unroll=2)
        def _(off):
            xv = t1[pl.ds(off, _L)]
            yv = t2[pl.ds(off, _L)]
            zv = t3[pl.ds(off, _L)]
            t4[pl.ds(off, _L)] = (xv * xv + yv * yv) + zv * zv
            t1[pl.ds(off, _L)] = _round_bf16(xv)
            t2[pl.ds(off, _L)] = _round_bf16(yv)
            t3[pl.ds(off, _L)] = _round_bf16(zv)

        def center_body(p, _):
            cxv = plsc.load_gather(cball, [jnp.full((_L,), p, jnp.int32)])
            cyv = plsc.load_gather(cball,
                                   [jnp.full((_L,), cpw + p, jnp.int32)])
            czv = plsc.load_gather(cball,
                                   [jnp.full((_L,), 2 * cpw + p, jnp.int32)])
            n2v = (cxv * cxv + cyv * cyv) + czv * czv
            cxb = _round_bf16(cxv)
            cyb = _round_bf16(cyv)
            czb = _round_bf16(czv)

            def cond(carry):
                j, cnt = carry
                return jnp.logical_and(cnt < _NSAMPLE, j < n_blocks)

            def wbody(carry):
                j, cnt = carry
                base = j * (_L * _BLK)
                masks = []
                sums = []
                for k in range(_BLK):
                    off = base + k * _L
                    xv = t1[pl.ds(off, _L)]
                    yv = t2[pl.ds(off, _L)]
                    zv = t3[pl.ds(off, _L)]
                    x2v = t4[pl.ds(off, _L)]
                    dot = (cxb * xv + cyb * yv) + czb * zv
                    d2 = (n2v + x2v) - 2.0 * dot
                    m = d2 < r2
                    masks.append(m)
                    sums.append(jnp.sum(m.astype(jnp.int32)))
                off_k = cnt
                for k in range(_BLK):
                    plsc.store_compressed(cbuf.at[pl.ds(off_k, _L)],
                                          iota + (base + k * _L),
                                          mask=masks[k])
                    off_k = off_k + sums[k]
                return (j + 1, off_k)

            _, cnt = lax.while_loop(cond, wbody,
                                    (jnp.int32(0), jnp.int32(0)))
            total = jnp.minimum(cnt, _NSAMPLE)
            tsplat = jnp.full((_L,), total, jnp.int32)
            firstv = plsc.load_gather(cbuf, [zero16])
            firstv = jnp.where(tsplat > 0, firstv, 0)
            for k in range(NS4):
                sl = iota + k * _L
                v = cbuf[pl.ds(k * _L, _L)]
                v = jnp.where(sl < tsplat, v, firstv)
                plsc.store_scatter(idx_all, [sl * cpw + p], v)
            return 0
        lax.fori_loop(0, cpw, center_body, 0)

        def start_table(c, tab, sem):
            @pl.when(c < 3)
            def _():
                pltpu.async_copy(xyzt.at[3 * b + c], tab, sem)

            @pl.when(jnp.logical_and(c >= 3, c < NCH))
            def _():
                pltpu.async_copy(feat.at[b, c - 3], tab, sem)

        def wait_table(tab, sem):
            pltpu.make_async_copy(feat.at[b, 0], tab, sem).wait()

        PC = cpw // _L

        def gather_chan(c, tab, stg):
            @pl.when(c < 3)
            def _():
                @plsc.parallel_loop(0, _NSAMPLE, unroll=2)
                def _(s):
                    for pc in range(PC):
                        o = s * cpw + pc * _L
                        idxv = idx_all[pl.ds(o, _L)]
                        vals = plsc.load_gather(tab, [idxv])
                        cv = cball[pl.ds(c * cpw + pc * _L, _L)]
                        stg[s, pl.ds(pc * _L, _L)] = vals - cv

            @pl.when(c >= 3)
            def _():
                @plsc.parallel_loop(0, _NSAMPLE, unroll=2)
                def _(s):
                    for pc in range(PC):
                        o = s * cpw + pc * _L
                        idxv = idx_all[pl.ds(o, _L)]
                        stg[s, pl.ds(pc * _L, _L)] = plsc.load_gather(
                            tab, [idxv])

        def half(i, c, tab, stg, semT, semW):
            wait_table(tab, semT)

            @pl.when(i > 0)
            def _():
                pltpu.make_async_copy(
                    stg, out.at[b, c - 2, :, pl.ds(p0, cpw)], semW).wait()
            gather_chan(c, tab, stg)
            start_table(c + 2, tab, semT)
            pltpu.async_copy(stg, out.at[b, c, :, pl.ds(p0, cpw)], semW)

        start_table(jnp.int32(0), t1, semTA)
        start_table(jnp.int32(1), t2, semTB)

        npairs = NCH // 2

        def chan_pair(i, _):
            half(i, 2 * i, t1, sA, semTA, semWA)
            half(i, 2 * i + 1, t2, sB, semTB, semWB)
            return 0
        lax.fori_loop(0, npairs, chan_pair, 0)

        if NCH % 2:
            cR = jnp.int32(NCH - 1)
            wait_table(t1, semTA)
            pltpu.make_async_copy(
                sA, out.at[b, cR - 2, :, pl.ds(p0, cpw)], semWA).wait()
            gather_chan(cR, t1, sA)
            pltpu.async_copy(sA, out.at[b, cR, :, pl.ds(p0, cpw)], semWA)
            pltpu.make_async_copy(
                sB, out.at[b, cR - 1, :, pl.ds(p0, cpw)], semWB).wait()
            pltpu.make_async_copy(
                sA, out.at[b, cR, :, pl.ds(p0, cpw)], semWA).wait()
        else:
            pltpu.make_async_copy(
                sA, out.at[b, NCH - 2, :, pl.ds(p0, cpw)], semWA).wait()
            pltpu.make_async_copy(
                sB, out.at[b, NCH - 1, :, pl.ds(p0, cpw)], semWB).wait()

    mesh = plsc.VectorSubcoreMesh(core_axis_name="c", subcore_axis_name="s",
                                  num_cores=_NC, num_subcores=_NS)
    return pl.kernel(
        body,
        out_type=jax.ShapeDtypeStruct((B, NCH, _NSAMPLE, P), jnp.float32),
        mesh=mesh,
        compiler_params=pltpu.CompilerParams(needs_layout_passes=False),
        scratch_types=[
            pltpu.VMEM((N,), jnp.float32),
            pltpu.VMEM((N,), jnp.float32),
            pltpu.VMEM((N,), jnp.float32),
            pltpu.VMEM((N,), jnp.float32),
            pltpu.VMEM((3 * cpw,), jnp.float32),
            pltpu.VMEM((_NSAMPLE + _L * _BLK,), jnp.int32),
            pltpu.VMEM((_NSAMPLE * cpw,), jnp.int32),
            pltpu.VMEM((_NSAMPLE, cpw), jnp.float32),
            pltpu.VMEM((_NSAMPLE, cpw), jnp.float32),
            pltpu.SemaphoreType.DMA,
            pltpu.SemaphoreType.DMA,
            pltpu.SemaphoreType.DMA,
            pltpu.SemaphoreType.DMA,
        ],
    )


def kernel(xyz, new_xyz, features):
    B, N, _ = xyz.shape
    P = new_xyz.shape[1]
    C = features.shape[1]
    xyzt = jnp.transpose(xyz, (0, 2, 1)).reshape(B * 3, N)
    cxs = new_xyz[:, :, 0]
    cys = new_xyz[:, :, 1]
    czs = new_xyz[:, :, 2]
    run = _make_sc_kernel(B, N, P, C)
    o = run(xyzt, cxs, cys, czs, features)
    return jnp.transpose(o, (0, 1, 3, 2))


# speedup vs baseline: 71.8718x; 1.0002x over previous
import jax
import jax.numpy as jnp
from jax import lax
from jax.experimental import pallas as pl
from jax.experimental.pallas import tpu as pltpu
from jax.experimental.pallas import tpu_sc as plsc

_RADIUS = 0.2
_NSAMPLE = 64
_NC, _NS, _L = 2, 16, 16
_NW = _NC * _NS
_BLK = 4


def _round_bf16(v):
    u = plsc.bitcast(v, jnp.int32)
    r = lax.shift_right_logical(u, 16) & 1
    u2 = (u + 32767 + r) & jnp.int32(-65536)
    return plsc.bitcast(u2, jnp.float32)


def _make_sc_kernel(B, N, P, C):
    assert (B * P) % _NW == 0
    cpw = (B * P) // _NW
    wpb = _NW // B
    assert N % (_L * _BLK) == 0 and _NSAMPLE % _L == 0
    n_blocks = N // (_L * _BLK)
    r2 = jnp.float32(_RADIUS * _RADIUS)
    NS4 = _NSAMPLE // _L
    NCH = 3 + C

    def body(xyzt, cxs, cys, czs, feat, out,
             t1, t2, t3, t4, cball, cbuf, idx_all, sA, sB,
             semTA, semTB, semWA, semWB):
        wid = lax.axis_index("s") * _NC + lax.axis_index("c")
        b = wid // wpb
        p0 = (wid % wpb) * cpw

        pltpu.sync_copy(xyzt.at[3 * b + 0], t1)
        pltpu.sync_copy(xyzt.at[3 * b + 1], t2)
        pltpu.sync_copy(xyzt.at[3 * b + 2], t3)
        pltpu.sync_copy(cxs.at[b, pl.ds(p0, cpw)], cball.at[pl.ds(0, cpw)])
        pltpu.sync_copy(cys.at[b, pl.ds(p0, cpw)], cball.at[pl.ds(cpw, cpw)])
        pltpu.sync_copy(czs.at[b, pl.ds(p0, cpw)],
                        cball.at[pl.ds(2 * cpw, cpw)])

        iota = lax.iota(jnp.int32, _L)
        zero16 = jnp.zeros((_L,), jnp.int32)

        @plsc.parallel_loop(0, N, step=_L, unroll=2)
        def _(off):
            xv = t1[pl.ds(off, _L)]
            yv = t2[pl.ds(off, _L)]
            zv = t3[pl.ds(off, _L)]
            t4[pl.ds(off, _L)] = (xv * xv + yv * yv) + zv * zv
            t1[pl.ds(off, _L)] = _round_bf16(xv)
            t2[pl.ds(off, _L)] = _round_bf16(yv)
            t3[pl.ds(off, _L)] = _round_bf16(zv)

        scope_bq = jax.named_scope("bq")
        scope_bq.__enter__()

        def center_body(p, _):
            cxv = plsc.load_gather(cball, [jnp.full((_L,), p, jnp.int32)])
            cyv = plsc.load_gather(cball,
                                   [jnp.full((_L,), cpw + p, jnp.int32)])
            czv = plsc.load_gather(cball,
                                   [jnp.full((_L,), 2 * cpw + p, jnp.int32)])
            n2v = (cxv * cxv + cyv * cyv) + czv * czv
            cxb = _round_bf16(cxv)
            cyb = _round_bf16(cyv)
            czb = _round_bf16(czv)

            def cond(carry):
                j, cnt = carry
                return jnp.logical_and(cnt < _NSAMPLE, j < n_blocks)

            def wbody(carry):
                j, cnt = carry
                base = j * (_L * _BLK)
                masks = []
                sums = []
                for k in range(_BLK):
                    off = base + k * _L
                    xv = t1[pl.ds(off, _L)]
                    yv = t2[pl.ds(off, _L)]
                    zv = t3[pl.ds(off, _L)]
                    x2v = t4[pl.ds(off, _L)]
                    dot = (cxb * xv + cyb * yv) + czb * zv
                    d2 = (n2v + x2v) - 2.0 * dot
                    m = d2 < r2
                    masks.append(m)
                    sums.append(jnp.sum(m.astype(jnp.int32)))
                off_k = cnt
                for k in range(_BLK):
                    plsc.store_compressed(cbuf.at[pl.ds(off_k, _L)],
                                          iota + (base + k * _L),
                                          mask=masks[k])
                    off_k = off_k + sums[k]
                return (j + 1, off_k)

            _, cnt = lax.while_loop(cond, wbody,
                                    (jnp.int32(0), jnp.int32(0)))
            total = jnp.minimum(cnt, _NSAMPLE)
            tsplat = jnp.full((_L,), total, jnp.int32)
            firstv = plsc.load_gather(cbuf, [zero16])
            firstv = jnp.where(tsplat > 0, firstv, 0)
            for k in range(NS4):
                sl = iota + k * _L
                v = cbuf[pl.ds(k * _L, _L)]
                v = jnp.where(sl < tsplat, v, firstv)
                plsc.store_scatter(idx_all, [sl * cpw + p], v)
            return 0
        lax.fori_loop(0, cpw, center_body, 0)
        scope_bq.__exit__(None, None, None)
        scope_g = jax.named_scope("gather")
        scope_g.__enter__()

        def start_table(c, tab, sem):
            @pl.when(c < 3)
            def _():
                pltpu.async_copy(xyzt.at[3 * b + c], tab, sem)

            @pl.when(jnp.logical_and(c >= 3, c < NCH))
            def _():
                pltpu.async_copy(feat.at[b, c - 3], tab, sem)

        def wait_table(tab, sem):
            pltpu.make_async_copy(feat.at[b, 0], tab, sem).wait()

        PC = cpw // _L

        def gather_chan(c, tab, stg):
            @pl.when(c < 3)
            def _():
                @plsc.parallel_loop(0, _NSAMPLE, unroll=2)
                def _(s):
                    for pc in range(PC):
                        o = s * cpw + pc * _L
                        idxv = idx_all[pl.ds(o, _L)]
                        vals = plsc.load_gather(tab, [idxv])
                        cv = cball[pl.ds(c * cpw + pc * _L, _L)]
                        stg[s, pl.ds(pc * _L, _L)] = vals - cv

            @pl.when(c >= 3)
            def _():
                @plsc.parallel_loop(0, _NSAMPLE, unroll=2)
                def _(s):
                    for pc in range(PC):
                        o = s * cpw + pc * _L
                        idxv = idx_all[pl.ds(o, _L)]
                        stg[s, pl.ds(pc * _L, _L)] = plsc.load_gather(
                            tab, [idxv])

        def half(i, c, tab, stg, semT, semW):
            wait_table(tab, semT)

            @pl.when(i > 0)
            def _():
                pltpu.make_async_copy(
                    stg, out.at[b, c - 2, :, pl.ds(p0, cpw)], semW).wait()
            gather_chan(c, tab, stg)
            start_table(c + 2, tab, semT)
            pltpu.async_copy(stg, out.at[b, c, :, pl.ds(p0, cpw)], semW)

        start_table(jnp.int32(0), t1, semTA)
        start_table(jnp.int32(1), t2, semTB)

        npairs = NCH // 2

        def chan_pair(i, _):
            half(i, 2 * i, t1, sA, semTA, semWA)
            half(i, 2 * i + 1, t2, sB, semTB, semWB)
            return 0
        lax.fori_loop(0, npairs, chan_pair, 0)

        if NCH % 2:
            cR = jnp.int32(NCH - 1)
            wait_table(t1, semTA)
            pltpu.make_async_copy(
                sA, out.at[b, cR - 2, :, pl.ds(p0, cpw)], semWA).wait()
            gather_chan(cR, t1, sA)
            pltpu.async_copy(sA, out.at[b, cR, :, pl.ds(p0, cpw)], semWA)
            pltpu.make_async_copy(
                sB, out.at[b, cR - 1, :, pl.ds(p0, cpw)], semWB).wait()
            pltpu.make_async_copy(
                sA, out.at[b, cR, :, pl.ds(p0, cpw)], semWA).wait()
        else:
            pltpu.make_async_copy(
                sA, out.at[b, NCH - 2, :, pl.ds(p0, cpw)], semWA).wait()
            pltpu.make_async_copy(
                sB, out.at[b, NCH - 1, :, pl.ds(p0, cpw)], semWB).wait()
        scope_g.__exit__(None, None, None)

    mesh = plsc.VectorSubcoreMesh(core_axis_name="c", subcore_axis_name="s",
                                  num_cores=_NC, num_subcores=_NS)
    return pl.kernel(
        body,
        out_type=jax.ShapeDtypeStruct((B, NCH, _NSAMPLE, P), jnp.float32),
        mesh=mesh,
        compiler_params=pltpu.CompilerParams(needs_layout_passes=False),
        scratch_types=[
            pltpu.VMEM((N,), jnp.float32),
            pltpu.VMEM((N,), jnp.float32),
            pltpu.VMEM((N,), jnp.float32),
            pltpu.VMEM((N,), jnp.float32),
            pltpu.VMEM((3 * cpw,), jnp.float32),
            pltpu.VMEM((_NSAMPLE + _L * _BLK,), jnp.int32),
            pltpu.VMEM((_NSAMPLE * cpw,), jnp.int32),
            pltpu.VMEM((_NSAMPLE, cpw), jnp.float32),
            pltpu.VMEM((_NSAMPLE, cpw), jnp.float32),
            pltpu.SemaphoreType.DMA,
            pltpu.SemaphoreType.DMA,
            pltpu.SemaphoreType.DMA,
            pltpu.SemaphoreType.DMA,
        ],
    )


def kernel(xyz, new_xyz, features):
    B, N, _ = xyz.shape
    P = new_xyz.shape[1]
    C = features.shape[1]
    xyzt = jnp.transpose(xyz, (0, 2, 1)).reshape(B * 3, N)
    cxs = new_xyz[:, :, 0]
    cys = new_xyz[:, :, 1]
    czs = new_xyz[:, :, 2]
    run = _make_sc_kernel(B, N, P, C)
    o = run(xyzt, cxs, cys, czs, features)
    return jnp.transpose(o, (0, 1, 3, 2))


# speedup vs baseline: 72.7424x; 1.0121x over previous
import jax
import jax.numpy as jnp
from jax import lax
from jax.experimental import pallas as pl
from jax.experimental.pallas import tpu as pltpu
from jax.experimental.pallas import tpu_sc as plsc

_RADIUS = 0.2
_NSAMPLE = 64
_NC, _NS, _L = 2, 16, 16
_NW = _NC * _NS
_BLK = 4


def _round_bf16(v):
    u = plsc.bitcast(v, jnp.int32)
    r = lax.shift_right_logical(u, 16) & 1
    u2 = (u + 32767 + r) & jnp.int32(-65536)
    return plsc.bitcast(u2, jnp.float32)


def _make_sc_kernel(B, N, P, C):
    assert (B * P) % _NW == 0
    cpw = (B * P) // _NW
    wpb = _NW // B
    assert N % (_L * _BLK) == 0 and _NSAMPLE % _L == 0 and cpw % 2 == 0
    n_blocks = N // (_L * _BLK)
    r2 = jnp.float32(_RADIUS * _RADIUS)
    NS4 = _NSAMPLE // _L
    NCH = 3 + C
    PC = cpw // _L

    def body(xyzt, cxs, cys, czs, feat, out,
             t1, t2, t3, t4, cball, cbufA, cbufB, idx_all, sA, sB,
             semTA, semTB, semWA, semWB):
        wid = lax.axis_index("s") * _NC + lax.axis_index("c")
        b = wid // wpb
        p0 = (wid % wpb) * cpw

        pltpu.sync_copy(xyzt.at[3 * b + 0], t1)
        pltpu.sync_copy(xyzt.at[3 * b + 1], t2)
        pltpu.sync_copy(xyzt.at[3 * b + 2], t3)
        pltpu.sync_copy(cxs.at[b, pl.ds(p0, cpw)], cball.at[pl.ds(0, cpw)])
        pltpu.sync_copy(cys.at[b, pl.ds(p0, cpw)], cball.at[pl.ds(cpw, cpw)])
        pltpu.sync_copy(czs.at[b, pl.ds(p0, cpw)],
                        cball.at[pl.ds(2 * cpw, cpw)])

        iota = lax.iota(jnp.int32, _L)
        zero16 = jnp.zeros((_L,), jnp.int32)

        @plsc.parallel_loop(0, N, step=_L, unroll=2)
        def _(off):
            xv = t1[pl.ds(off, _L)]
            yv = t2[pl.ds(off, _L)]
            zv = t3[pl.ds(off, _L)]
            t4[pl.ds(off, _L)] = (xv * xv + yv * yv) + zv * zv
            t1[pl.ds(off, _L)] = _round_bf16(xv)
            t2[pl.ds(off, _L)] = _round_bf16(yv)
            t3[pl.ds(off, _L)] = _round_bf16(zv)

        scope_bq = jax.named_scope("bq")
        scope_bq.__enter__()

        def load_center(p):
            cxv = plsc.load_gather(cball, [jnp.full((_L,), p, jnp.int32)])
            cyv = plsc.load_gather(cball,
                                   [jnp.full((_L,), cpw + p, jnp.int32)])
            czv = plsc.load_gather(cball,
                                   [jnp.full((_L,), 2 * cpw + p, jnp.int32)])
            n2v = (cxv * cxv + cyv * cyv) + czv * czv
            return _round_bf16(cxv), _round_bf16(cyv), _round_bf16(czv), n2v

        def pad_one(p, cbuf, cnt):
            total = jnp.minimum(cnt, _NSAMPLE)
            tsplat = jnp.full((_L,), total, jnp.int32)
            firstv = plsc.load_gather(cbuf, [zero16])
            firstv = jnp.where(tsplat > 0, firstv, 0)
            for k in range(NS4):
                sl = iota + k * _L
                v = cbuf[pl.ds(k * _L, _L)]
                v = jnp.where(sl < tsplat, v, firstv)
                plsc.store_scatter(idx_all, [sl * cpw + p], v)

        def center_body(pp, _):
            pa = 2 * pp
            pb = pa + 1
            axb, ayb, azb, an2 = load_center(pa)
            bxb, byb, bzb, bn2 = load_center(pb)

            def cond(carry):
                j, ca, cb = carry
                return jnp.logical_and(
                    jnp.logical_or(ca < _NSAMPLE, cb < _NSAMPLE),
                    j < n_blocks)

            def wbody(carry):
                j, ca, cb = carry
                base = j * (_L * _BLK)
                masks_a, sums_a, masks_b, sums_b = [], [], [], []
                for k in range(_BLK):
                    off = base + k * _L
                    xv = t1[pl.ds(off, _L)]
                    yv = t2[pl.ds(off, _L)]
                    zv = t3[pl.ds(off, _L)]
                    x2v = t4[pl.ds(off, _L)]
                    da = (axb * xv + ayb * yv) + azb * zv
                    ma = ((an2 + x2v) - 2.0 * da) < r2
                    db = (bxb * xv + byb * yv) + bzb * zv
                    mb = ((bn2 + x2v) - 2.0 * db) < r2
                    masks_a.append(ma)
                    sums_a.append(jnp.sum(ma.astype(jnp.int32)))
                    masks_b.append(mb)
                    sums_b.append(jnp.sum(mb.astype(jnp.int32)))
                oa = jnp.minimum(ca, _NSAMPLE)
                ob = jnp.minimum(cb, _NSAMPLE)
                for k in range(_BLK):
                    off = base + k * _L
                    plsc.store_compressed(cbufA.at[pl.ds(oa, _L)],
                                          iota + off, mask=masks_a[k])
                    plsc.store_compressed(cbufB.at[pl.ds(ob, _L)],
                                          iota + off, mask=masks_b[k])
                    oa = oa + sums_a[k]
                    ob = ob + sums_b[k]
                ca = ca + sums_a[0] + sums_a[1] + sums_a[2] + sums_a[3]
                cb = cb + sums_b[0] + sums_b[1] + sums_b[2] + sums_b[3]
                return (j + 1, ca, cb)

            _, ca, cb = lax.while_loop(
                cond, wbody, (jnp.int32(0), jnp.int32(0), jnp.int32(0)))
            pad_one(pa, cbufA, ca)
            pad_one(pb, cbufB, cb)
            return 0
        lax.fori_loop(0, cpw // 2, center_body, 0)
        scope_bq.__exit__(None, None, None)
        scope_g = jax.named_scope("gather")
        scope_g.__enter__()

        def start_table(c, tab, sem):
            @pl.when(c < 3)
            def _():
                pltpu.async_copy(xyzt.at[3 * b + c], tab, sem)

            @pl.when(jnp.logical_and(c >= 3, c < NCH))
            def _():
                pltpu.async_copy(feat.at[b, c - 3], tab, sem)

        def wait_table(tab, sem):
            pltpu.make_async_copy(feat.at[b, 0], tab, sem).wait()

        def wait_wb(stg, c, sem):
            pltpu.make_async_copy(
                stg, out.at[b, c, :, pl.ds(p0, cpw)], sem).wait()

        def start_wb(stg, c, sem):
            pltpu.async_copy(stg, out.at[b, c, :, pl.ds(p0, cpw)], sem)

        def gather_pair(cA, cB, subA, subB):
            @plsc.parallel_loop(0, _NSAMPLE, unroll=2)
            def _(s):
                for pc in range(PC):
                    o = s * cpw + pc * _L
                    idxv = idx_all[pl.ds(o, _L)]
                    va = plsc.load_gather(t1, [idxv])
                    if subA:
                        va = va - cball[pl.ds(cA * cpw + pc * _L, _L)]
                    sA[s, pl.ds(pc * _L, _L)] = va
                    if cB is not None:
                        vb = plsc.load_gather(t2, [idxv])
                        if subB:
                            vb = vb - cball[pl.ds(cB * cpw + pc * _L, _L)]
                        sB[s, pl.ds(pc * _L, _L)] = vb

        start_table(jnp.int32(0), t1, semTA)
        start_table(jnp.int32(1), t2, semTB)

        wait_table(t1, semTA)
        wait_table(t2, semTB)
        gather_pair(0, 1, True, True)
        start_table(jnp.int32(2), t1, semTA)
        start_table(jnp.int32(3), t2, semTB)
        start_wb(sA, jnp.int32(0), semWA)
        start_wb(sB, jnp.int32(1), semWB)

        wait_table(t1, semTA)
        wait_table(t2, semTB)
        wait_wb(sA, jnp.int32(0), semWA)
        wait_wb(sB, jnp.int32(1), semWB)
        gather_pair(2, 3, True, False)
        start_table(jnp.int32(4), t1, semTA)
        start_table(jnp.int32(5), t2, semTB)
        start_wb(sA, jnp.int32(2), semWA)
        start_wb(sB, jnp.int32(3), semWB)

        def chan_pair(i, _):
            cA = 2 * i
            cB = cA + 1
            wait_table(t1, semTA)
            wait_table(t2, semTB)
            wait_wb(sA, cA - 2, semWA)
            wait_wb(sB, cB - 2, semWB)
            gather_pair(cA, cB, False, False)
            start_table(cA + 2, t1, semTA)
            start_table(cB + 2, t2, semTB)
            start_wb(sA, cA, semWA)
            start_wb(sB, cB, semWB)
            return 0
        lax.fori_loop(2, NCH // 2, chan_pair, 0)

        if NCH % 2:
            cR = jnp.int32(NCH - 1)
            wait_table(t1, semTA)
            wait_wb(sA, cR - 2, semWA)
            gather_pair(cR, None, False, False)
            start_wb(sA, cR, semWA)
            wait_wb(sB, cR - 1, semWB)
            wait_wb(sA, cR, semWA)
        else:
            wait_wb(sA, jnp.int32(NCH - 2), semWA)
            wait_wb(sB, jnp.int32(NCH - 1), semWB)
        scope_g.__exit__(None, None, None)

    mesh = plsc.VectorSubcoreMesh(core_axis_name="c", subcore_axis_name="s",
                                  num_cores=_NC, num_subcores=_NS)
    return pl.kernel(
        body,
        out_type=jax.ShapeDtypeStruct((B, NCH, _NSAMPLE, P), jnp.float32),
        mesh=mesh,
        compiler_params=pltpu.CompilerParams(needs_layout_passes=False),
        scratch_types=[
            pltpu.VMEM((N,), jnp.float32),
            pltpu.VMEM((N,), jnp.float32),
            pltpu.VMEM((N,), jnp.float32),
            pltpu.VMEM((N,), jnp.float32),
            pltpu.VMEM((3 * cpw,), jnp.float32),
            pltpu.VMEM((2 * _NSAMPLE + _L * _BLK,), jnp.int32),
            pltpu.VMEM((2 * _NSAMPLE + _L * _BLK,), jnp.int32),
            pltpu.VMEM((_NSAMPLE * cpw,), jnp.int32),
            pltpu.VMEM((_NSAMPLE, cpw), jnp.float32),
            pltpu.VMEM((_NSAMPLE, cpw), jnp.float32),
            pltpu.SemaphoreType.DMA,
            pltpu.SemaphoreType.DMA,
            pltpu.SemaphoreType.DMA,
            pltpu.SemaphoreType.DMA,
        ],
    )


def kernel(xyz, new_xyz, features):
    B, N, _ = xyz.shape
    P = new_xyz.shape[1]
    C = features.shape[1]
    xyzt = jnp.transpose(xyz, (0, 2, 1)).reshape(B * 3, N)
    cxs = new_xyz[:, :, 0]
    cys = new_xyz[:, :, 1]
    czs = new_xyz[:, :, 2]
    run = _make_sc_kernel(B, N, P, C)
    o = run(xyzt, cxs, cys, czs, features)
    return jnp.transpose(o, (0, 1, 3, 2))


# speedup vs baseline: 73.3106x; 1.0078x over previous
import jax
import jax.numpy as jnp
from jax import lax
from jax.experimental import pallas as pl
from jax.experimental.pallas import tpu as pltpu
from jax.experimental.pallas import tpu_sc as plsc

_RADIUS = 0.2
_NSAMPLE = 64
_NC, _NS, _L = 2, 16, 16
_NW = _NC * _NS
_BLK = 4


def _round_bf16(v):
    u = plsc.bitcast(v, jnp.int32)
    r = lax.shift_right_logical(u, 16) & 1
    u2 = (u + 32767 + r) & jnp.int32(-65536)
    return plsc.bitcast(u2, jnp.float32)


def _make_sc_kernel(B, N, P, C):
    assert (B * P) % _NW == 0
    cpw = (B * P) // _NW
    wpb = _NW // B
    assert N % (_L * _BLK) == 0 and _NSAMPLE % _L == 0 and cpw % 2 == 0
    n_blocks = N // (_L * _BLK)
    r2 = jnp.float32(_RADIUS * _RADIUS)
    NS4 = _NSAMPLE // _L
    NCH = 3 + C
    PC = cpw // _L

    def body(xyzt, cxs, cys, czs, feat, out,
             t1, t2, t3, t4, cball, cbufA, cbufB, idx_all, sA, sB,
             semTA, semTB, semWA, semWB):
        wid = lax.axis_index("s") * _NC + lax.axis_index("c")
        b = wid // wpb
        p0 = (wid % wpb) * cpw

        pltpu.sync_copy(xyzt.at[3 * b + 0], t1)
        pltpu.sync_copy(xyzt.at[3 * b + 1], t2)
        pltpu.sync_copy(xyzt.at[3 * b + 2], t3)
        pltpu.sync_copy(cxs.at[b, pl.ds(p0, cpw)], cball.at[pl.ds(0, cpw)])
        pltpu.sync_copy(cys.at[b, pl.ds(p0, cpw)], cball.at[pl.ds(cpw, cpw)])
        pltpu.sync_copy(czs.at[b, pl.ds(p0, cpw)],
                        cball.at[pl.ds(2 * cpw, cpw)])

        iota = lax.iota(jnp.int32, _L)
        zero16 = jnp.zeros((_L,), jnp.int32)

        @plsc.parallel_loop(0, N, step=_L, unroll=2)
        def _(off):
            xv = t1[pl.ds(off, _L)]
            yv = t2[pl.ds(off, _L)]
            zv = t3[pl.ds(off, _L)]
            t4[pl.ds(off, _L)] = (xv * xv + yv * yv) + zv * zv
            t1[pl.ds(off, _L)] = _round_bf16(xv)
            t2[pl.ds(off, _L)] = _round_bf16(yv)
            t3[pl.ds(off, _L)] = _round_bf16(zv)

        scope_bq = jax.named_scope("bq")
        scope_bq.__enter__()

        def load_center(p):
            cxv = plsc.load_gather(cball, [jnp.full((_L,), p, jnp.int32)])
            cyv = plsc.load_gather(cball,
                                   [jnp.full((_L,), cpw + p, jnp.int32)])
            czv = plsc.load_gather(cball,
                                   [jnp.full((_L,), 2 * cpw + p, jnp.int32)])
            n2v = (cxv * cxv + cyv * cyv) + czv * czv
            return _round_bf16(cxv), _round_bf16(cyv), _round_bf16(czv), n2v

        def pad_one(p, cbuf, cnt):
            total = jnp.minimum(cnt, _NSAMPLE)
            tsplat = jnp.full((_L,), total, jnp.int32)
            firstv = plsc.load_gather(cbuf, [zero16])
            firstv = jnp.where(tsplat > 0, firstv, 0)
            for k in range(NS4):
                sl = iota + k * _L
                v = cbuf[pl.ds(k * _L, _L)]
                v = jnp.where(sl < tsplat, v, firstv)
                plsc.store_scatter(idx_all, [sl * cpw + p], v)

        def center_body(pp, _):
            pa = 2 * pp
            pb = pa + 1
            axb, ayb, azb, an2 = load_center(pa)
            bxb, byb, bzb, bn2 = load_center(pb)

            def cond(carry):
                j, ca, cb = carry
                return jnp.logical_and(
                    jnp.logical_or(ca < _NSAMPLE, cb < _NSAMPLE),
                    j < n_blocks)

            def wbody(carry):
                j, ca, cb = carry
                base = j * (_L * _BLK)
                masks_a, sums_a, masks_b, sums_b = [], [], [], []
                for k in range(_BLK):
                    off = base + k * _L
                    xv = t1[pl.ds(off, _L)]
                    yv = t2[pl.ds(off, _L)]
                    zv = t3[pl.ds(off, _L)]
                    x2v = t4[pl.ds(off, _L)]
                    da = (axb * xv + ayb * yv) + azb * zv
                    ma = ((an2 + x2v) - 2.0 * da) < r2
                    db = (bxb * xv + byb * yv) + bzb * zv
                    mb = ((bn2 + x2v) - 2.0 * db) < r2
                    masks_a.append(ma)
                    sums_a.append(jnp.sum(ma.astype(jnp.int32)))
                    masks_b.append(mb)
                    sums_b.append(jnp.sum(mb.astype(jnp.int32)))
                oa = jnp.minimum(ca, _NSAMPLE)
                ob = jnp.minimum(cb, _NSAMPLE)
                for k in range(_BLK):
                    off = base + k * _L
                    plsc.store_compressed(cbufA.at[pl.ds(oa, _L)],
                                          iota + off, mask=masks_a[k])
                    plsc.store_compressed(cbufB.at[pl.ds(ob, _L)],
                                          iota + off, mask=masks_b[k])
                    oa = oa + sums_a[k]
                    ob = ob + sums_b[k]
                ca = ca + sums_a[0] + sums_a[1] + sums_a[2] + sums_a[3]
                cb = cb + sums_b[0] + sums_b[1] + sums_b[2] + sums_b[3]
                return (j + 1, ca, cb)

            _, ca, cb = lax.while_loop(
                cond, wbody, (jnp.int32(0), jnp.int32(0), jnp.int32(0)))
            pad_one(pa, cbufA, ca)
            pad_one(pb, cbufB, cb)
            return 0
        lax.fori_loop(0, cpw // 2, center_body, 0)
        scope_bq.__exit__(None, None, None)
        scope_g = jax.named_scope("gather")
        scope_g.__enter__()

        def start_table(c, tab, sem):
            @pl.when(c < 3)
            def _():
                pltpu.async_copy(xyzt.at[3 * b + c], tab, sem)

            @pl.when(jnp.logical_and(c >= 3, c < NCH))
            def _():
                pltpu.async_copy(feat.at[b, c - 3], tab, sem)

        def wait_table(tab, sem):
            pltpu.make_async_copy(feat.at[b, 0], tab, sem).wait()

        def wait_wb(stg, c, sem):
            pltpu.make_async_copy(
                stg, out.at[b, c, :, pl.ds(p0, cpw)], sem).wait()

        def start_wb(stg, c, sem):
            pltpu.async_copy(stg, out.at[b, c, :, pl.ds(p0, cpw)], sem)

        def gather_pair(cA, cB, subA, subB):
            @plsc.parallel_loop(0, _NSAMPLE, unroll=1)
            def _(s):
                for pc in range(PC):
                    o = s * cpw + pc * _L
                    idxv = idx_all[pl.ds(o, _L)]
                    va = plsc.load_gather(t1, [idxv])
                    if subA:
                        va = va - cball[pl.ds(cA * cpw + pc * _L, _L)]
                    sA[s, pl.ds(pc * _L, _L)] = va
                    if cB is not None:
                        vb = plsc.load_gather(t2, [idxv])
                        if subB:
                            vb = vb - cball[pl.ds(cB * cpw + pc * _L, _L)]
                        sB[s, pl.ds(pc * _L, _L)] = vb

        start_table(jnp.int32(0), t1, semTA)
        start_table(jnp.int32(1), t2, semTB)

        wait_table(t1, semTA)
        wait_table(t2, semTB)
        gather_pair(0, 1, True, True)
        start_table(jnp.int32(2), t1, semTA)
        start_table(jnp.int32(3), t2, semTB)
        start_wb(sA, jnp.int32(0), semWA)
        start_wb(sB, jnp.int32(1), semWB)

        wait_table(t1, semTA)
        wait_table(t2, semTB)
        wait_wb(sA, jnp.int32(0), semWA)
        wait_wb(sB, jnp.int32(1), semWB)
        gather_pair(2, 3, True, False)
        start_table(jnp.int32(4), t1, semTA)
        start_table(jnp.int32(5), t2, semTB)
        start_wb(sA, jnp.int32(2), semWA)
        start_wb(sB, jnp.int32(3), semWB)

        def chan_pair(i, _):
            cA = 2 * i
            cB = cA + 1
            wait_table(t1, semTA)
            wait_table(t2, semTB)
            wait_wb(sA, cA - 2, semWA)
            wait_wb(sB, cB - 2, semWB)
            gather_pair(cA, cB, False, False)
            start_table(cA + 2, t1, semTA)
            start_table(cB + 2, t2, semTB)
            start_wb(sA, cA, semWA)
            start_wb(sB, cB, semWB)
            return 0
        lax.fori_loop(2, NCH // 2, chan_pair, 0)

        if NCH % 2:
            cR = jnp.int32(NCH - 1)
            wait_table(t1, semTA)
            wait_wb(sA, cR - 2, semWA)
            gather_pair(cR, None, False, False)
            start_wb(sA, cR, semWA)
            wait_wb(sB, cR - 1, semWB)
            wait_wb(sA, cR, semWA)
        else:
            wait_wb(sA, jnp.int32(NCH - 2), semWA)
            wait_wb(sB, jnp.int32(NCH - 1), semWB)
        scope_g.__exit__(None, None, None)

    mesh = plsc.VectorSubcoreMesh(core_axis_name="c", subcore_axis_name="s",
                                  num_cores=_NC, num_subcores=_NS)
    return pl.kernel(
        body,
        out_type=jax.ShapeDtypeStruct((B, NCH, _NSAMPLE, P), jnp.float32),
        mesh=mesh,
        compiler_params=pltpu.CompilerParams(needs_layout_passes=False),
        scratch_types=[
            pltpu.VMEM((N,), jnp.float32),
            pltpu.VMEM((N,), jnp.float32),
            pltpu.VMEM((N,), jnp.float32),
            pltpu.VMEM((N,), jnp.float32),
            pltpu.VMEM((3 * cpw,), jnp.float32),
            pltpu.VMEM((2 * _NSAMPLE + _L * _BLK,), jnp.int32),
            pltpu.VMEM((2 * _NSAMPLE + _L * _BLK,), jnp.int32),
            pltpu.VMEM((_NSAMPLE * cpw,), jnp.int32),
            pltpu.VMEM((_NSAMPLE, cpw), jnp.float32),
            pltpu.VMEM((_NSAMPLE, cpw), jnp.float32),
            pltpu.SemaphoreType.DMA,
            pltpu.SemaphoreType.DMA,
            pltpu.SemaphoreType.DMA,
            pltpu.SemaphoreType.DMA,
        ],
    )


def kernel(xyz, new_xyz, features):
    B, N, _ = xyz.shape
    P = new_xyz.shape[1]
    C = features.shape[1]
    xyzt = jnp.transpose(xyz, (0, 2, 1)).reshape(B * 3, N)
    cxs = new_xyz[:, :, 0]
    cys = new_xyz[:, :, 1]
    czs = new_xyz[:, :, 2]
    run = _make_sc_kernel(B, N, P, C)
    o = run(xyzt, cxs, cys, czs, features)
    return jnp.transpose(o, (0, 1, 3, 2))


# speedup vs baseline: 82.3441x; 1.1232x over previous
import jax
import jax.numpy as jnp
from jax import lax
from jax.experimental import pallas as pl
from jax.experimental.pallas import tpu as pltpu
from jax.experimental.pallas import tpu_sc as plsc

_RADIUS = 0.2
_NSAMPLE = 64
_NC, _NS, _L = 2, 16, 16
_NW = _NC * _NS
_BLK = 4


def _round_bf16(v):
    u = plsc.bitcast(v, jnp.int32)
    r = lax.shift_right_logical(u, 16) & 1
    u2 = (u + 32767 + r) & jnp.int32(-65536)
    return plsc.bitcast(u2, jnp.float32)


def _make_sc_kernel(B, N, P, C):
    assert (B * P) % _NW == 0
    cpw = (B * P) // _NW
    wpb = _NW // B
    assert N % (_L * _BLK) == 0 and _NSAMPLE % _L == 0 and cpw % 2 == 0
    n_blocks = N // (_L * _BLK)
    r2 = jnp.float32(_RADIUS * _RADIUS)
    NS4 = _NSAMPLE // _L
    NCH = 3 + C
    PC = cpw // _L

    def body(xyzt, cxs, cys, czs, feat, out,
             t1, t2, t3, t4, cball, cbufA, cbufB, idx_all, sA, sB,
             semTA, semTB, semWA, semWB):
        wid = lax.axis_index("s") * _NC + lax.axis_index("c")
        b = wid // wpb
        p0 = (wid % wpb) * cpw

        pltpu.sync_copy(xyzt.at[3 * b + 0], t1)
        pltpu.sync_copy(xyzt.at[3 * b + 1], t2)
        pltpu.sync_copy(xyzt.at[3 * b + 2], t3)
        pltpu.sync_copy(cxs.at[b, pl.ds(p0, cpw)], cball.at[pl.ds(0, cpw)])
        pltpu.sync_copy(cys.at[b, pl.ds(p0, cpw)], cball.at[pl.ds(cpw, cpw)])
        pltpu.sync_copy(czs.at[b, pl.ds(p0, cpw)],
                        cball.at[pl.ds(2 * cpw, cpw)])

        iota = lax.iota(jnp.int32, _L)
        zero16 = jnp.zeros((_L,), jnp.int32)

        @plsc.parallel_loop(0, N, step=_L, unroll=2)
        def _(off):
            xv = t1[pl.ds(off, _L)]
            yv = t2[pl.ds(off, _L)]
            zv = t3[pl.ds(off, _L)]
            t4[pl.ds(off, _L)] = (xv * xv + yv * yv) + zv * zv
            t1[pl.ds(off, _L)] = _round_bf16(xv)
            t2[pl.ds(off, _L)] = _round_bf16(yv)
            t3[pl.ds(off, _L)] = _round_bf16(zv)

        scope_bq = jax.named_scope("bq")
        scope_bq.__enter__()

        def load_center(p):
            cxv = plsc.load_gather(cball, [jnp.full((_L,), p, jnp.int32)])
            cyv = plsc.load_gather(cball,
                                   [jnp.full((_L,), cpw + p, jnp.int32)])
            czv = plsc.load_gather(cball,
                                   [jnp.full((_L,), 2 * cpw + p, jnp.int32)])
            n2v = (cxv * cxv + cyv * cyv) + czv * czv
            return _round_bf16(cxv), _round_bf16(cyv), _round_bf16(czv), n2v

        def pad_one(p, cbuf, cnt):
            total = jnp.minimum(cnt, _NSAMPLE)
            tsplat = jnp.full((_L,), total, jnp.int32)
            firstv = plsc.load_gather(cbuf, [zero16])
            firstv = jnp.where(tsplat > 0, firstv, 0)
            for k in range(NS4):
                sl = iota + k * _L
                v = cbuf[pl.ds(k * _L, _L)]
                v = jnp.where(sl < tsplat, v, firstv)
                plsc.store_scatter(idx_all, [sl * cpw + p], v)

        def center_body(pp, _):
            pa = 2 * pp
            pb = pa + 1
            axb, ayb, azb, an2 = load_center(pa)
            bxb, byb, bzb, bn2 = load_center(pb)

            def cond(carry):
                j, ca, cb = carry
                return jnp.logical_and(
                    jnp.logical_or(ca < _NSAMPLE, cb < _NSAMPLE),
                    j < n_blocks)

            def wbody(carry):
                j, ca, cb = carry
                base = j * (_L * _BLK)
                masks_a, sums_a, masks_b, sums_b = [], [], [], []
                for k in range(_BLK):
                    off = base + k * _L
                    xv = t1[pl.ds(off, _L)]
                    yv = t2[pl.ds(off, _L)]
                    zv = t3[pl.ds(off, _L)]
                    x2v = t4[pl.ds(off, _L)]
                    da = (axb * xv + ayb * yv) + azb * zv
                    ma = ((an2 + x2v) - 2.0 * da) < r2
                    db = (bxb * xv + byb * yv) + bzb * zv
                    mb = ((bn2 + x2v) - 2.0 * db) < r2
                    masks_a.append(ma)
                    sums_a.append(jnp.sum(ma.astype(jnp.int32)))
                    masks_b.append(mb)
                    sums_b.append(jnp.sum(mb.astype(jnp.int32)))
                oa = jnp.minimum(ca, _NSAMPLE)
                ob = jnp.minimum(cb, _NSAMPLE)
                for k in range(_BLK):
                    off = base + k * _L
                    plsc.store_compressed(cbufA.at[pl.ds(oa, _L)],
                                          iota + off, mask=masks_a[k])
                    plsc.store_compressed(cbufB.at[pl.ds(ob, _L)],
                                          iota + off, mask=masks_b[k])
                    oa = oa + sums_a[k]
                    ob = ob + sums_b[k]
                ca = ca + sums_a[0] + sums_a[1] + sums_a[2] + sums_a[3]
                cb = cb + sums_b[0] + sums_b[1] + sums_b[2] + sums_b[3]
                return (j + 1, ca, cb)

            _, ca, cb = lax.while_loop(
                cond, wbody, (jnp.int32(0), jnp.int32(0), jnp.int32(0)))
            pad_one(pa, cbufA, ca)
            pad_one(pb, cbufB, cb)
            return 0
        lax.fori_loop(0, cpw // 2, center_body, 0)
        scope_bq.__exit__(None, None, None)
        scope_g = jax.named_scope("gather")
        scope_g.__enter__()

        def start_table(c, tab, sem):
            @pl.when(c < 3)
            def _():
                pltpu.async_copy(xyzt.at[3 * b + c], tab, sem)

            @pl.when(jnp.logical_and(c >= 3, c < NCH))
            def _():
                pltpu.async_copy(feat.at[b, c - 3], tab, sem)

        def wait_table(tab, sem):
            pltpu.make_async_copy(feat.at[b, 0], tab, sem).wait()

        def wait_wb(stg, c, sem):
            pltpu.make_async_copy(
                stg, out.at[b, c, :, pl.ds(p0, cpw)], sem).wait()

        def start_wb(stg, c, sem):
            pltpu.async_copy(stg, out.at[b, c, :, pl.ds(p0, cpw)], sem)

        def gather_chan(c, tab, stg):
            @pl.when(c < 3)
            def _():
                @plsc.parallel_loop(0, _NSAMPLE, unroll=2)
                def _(s):
                    for pc in range(PC):
                        o = s * cpw + pc * _L
                        idxv = idx_all[pl.ds(o, _L)]
                        vals = plsc.load_gather(tab, [idxv])
                        cv = cball[pl.ds(c * cpw + pc * _L, _L)]
                        stg[s, pl.ds(pc * _L, _L)] = vals - cv

            @pl.when(c >= 3)
            def _():
                @plsc.parallel_loop(0, _NSAMPLE, unroll=2)
                def _(s):
                    for pc in range(PC):
                        o = s * cpw + pc * _L
                        idxv = idx_all[pl.ds(o, _L)]
                        stg[s, pl.ds(pc * _L, _L)] = plsc.load_gather(
                            tab, [idxv])

        def half(i, c, tab, stg, semT, semW):
            wait_table(tab, semT)

            @pl.when(i > 0)
            def _():
                wait_wb(stg, c - 2, semW)
            gather_chan(c, tab, stg)
            start_table(c + 2, tab, semT)
            start_wb(stg, c, semW)

        start_table(jnp.int32(0), t1, semTA)
        start_table(jnp.int32(1), t2, semTB)

        def chan_pair(i, _):
            half(i, 2 * i, t1, sA, semTA, semWA)
            half(i, 2 * i + 1, t2, sB, semTB, semWB)
            return 0
        lax.fori_loop(0, NCH // 2, chan_pair, 0)

        if NCH % 2:
            cR = jnp.int32(NCH - 1)
            wait_table(t1, semTA)
            wait_wb(sA, cR - 2, semWA)
            gather_chan(cR, t1, sA)
            start_wb(sA, cR, semWA)
            wait_wb(sB, cR - 1, semWB)
            wait_wb(sA, cR, semWA)
        else:
            wait_wb(sA, jnp.int32(NCH - 2), semWA)
            wait_wb(sB, jnp.int32(NCH - 1), semWB)
        scope_g.__exit__(None, None, None)

    mesh = plsc.VectorSubcoreMesh(core_axis_name="c", subcore_axis_name="s",
                                  num_cores=_NC, num_subcores=_NS)
    return pl.kernel(
        body,
        out_type=jax.ShapeDtypeStruct((B, NCH, _NSAMPLE, P), jnp.float32),
        mesh=mesh,
        compiler_params=pltpu.CompilerParams(needs_layout_passes=False),
        scratch_types=[
            pltpu.VMEM((N,), jnp.float32),
            pltpu.VMEM((N,), jnp.float32),
            pltpu.VMEM((N,), jnp.float32),
            pltpu.VMEM((N,), jnp.float32),
            pltpu.VMEM((3 * cpw,), jnp.float32),
            pltpu.VMEM((2 * _NSAMPLE + _L * _BLK,), jnp.int32),
            pltpu.VMEM((2 * _NSAMPLE + _L * _BLK,), jnp.int32),
            pltpu.VMEM((_NSAMPLE * cpw,), jnp.int32),
            pltpu.VMEM((_NSAMPLE, cpw), jnp.float32),
            pltpu.VMEM((_NSAMPLE, cpw), jnp.float32),
            pltpu.SemaphoreType.DMA,
            pltpu.SemaphoreType.DMA,
            pltpu.SemaphoreType.DMA,
            pltpu.SemaphoreType.DMA,
        ],
    )


def kernel(xyz, new_xyz, features):
    B, N, _ = xyz.shape
    P = new_xyz.shape[1]
    C = features.shape[1]
    xyzt = jnp.transpose(xyz, (0, 2, 1)).reshape(B * 3, N)
    cxs = new_xyz[:, :, 0]
    cys = new_xyz[:, :, 1]
    czs = new_xyz[:, :, 2]
    run = _make_sc_kernel(B, N, P, C)
    o = run(xyzt, cxs, cys, czs, features)
    return jnp.transpose(o, (0, 1, 3, 2))
